# baseline (speedup 1.0000x reference)
import functools

import jax
import jax.numpy as jnp
from jax import lax
from jax.experimental import pallas as pl
from jax.experimental.pallas import tpu as pltpu
from jax.experimental.pallas import tpu_sc as plsc

BATCH = 16384
EMBED = 32
ROWS = 1000000
DPAD = 7840
STRIDE = DPAD * 128
FLAT = EMBED * STRIDE
NC = 2
NS = 16
NW = NC * NS
BPW = BATCH // NW
CHUNK = 128
NCHUNK = BPW // CHUNK
CB = 32768


def _restage_body(inT_ref, out_ref):
    out_ref[...] = inT_ref[...].reshape(EMBED, CB // 128, 128)


def _restage(tT):
    return pl.pallas_call(
        _restage_body,
        grid=(pl.cdiv(ROWS, CB),),
        in_specs=[pl.BlockSpec((EMBED, CB), lambda j: (0, j))],
        out_specs=pl.BlockSpec((EMBED, CB // 128, 128), lambda j: (0, j, 0)),
        out_shape=jax.ShapeDtypeStruct((EMBED, DPAD, 128), jnp.float32),
    )(tT)


@functools.lru_cache(maxsize=1)
def _make_sc_gather():
    mesh = plsc.VectorSubcoreMesh(core_axis_name="c", subcore_axis_name="s")

    @functools.partial(
        pl.kernel,
        mesh=mesh,
        compiler_params=pltpu.CompilerParams(use_tc_tiling_on_sc=False),
        out_type=jax.ShapeDtypeStruct((EMBED, BATCH), jnp.float32),
        scratch_types=[
            pltpu.VMEM((BPW,), jnp.int32),
            pltpu.VMEM((EMBED, BPW), jnp.int32),
            pltpu.VMEM((EMBED, BPW), jnp.float32),
            pltpu.SemaphoreType.DMA,
        ],
    )
    def _sc_gather(idx_hbm, flat_hbm, eT_out, idx_v, fidx_v, rowsT_v, sem):
        wid = lax.axis_index("s") * NC + lax.axis_index("c")
        base = wid * BPW
        pltpu.sync_copy(idx_hbm.at[pl.ds(base, BPW)], idx_v)

        def fill_row(d, _):
            bv = jnp.full((16,), 0, jnp.int32) + d * STRIDE
            for k in range(BPW // 16):
                sl = pl.ds(k * 16, 16)
                fidx_v[d, sl] = idx_v[sl] + bv
            return 0

        lax.fori_loop(0, EMBED, fill_row, 0)

        copies = []
        for d in range(EMBED):
            for c in range(NCHUNK):
                sl = pl.ds(c * CHUNK, CHUNK)
                copies.append(pltpu.async_copy(
                    flat_hbm.at[fidx_v.at[d, sl]], rowsT_v.at[d, sl], sem))
        for cp in copies:
            cp.wait()

        pltpu.sync_copy(rowsT_v, eT_out.at[:, pl.ds(base, BPW)])

    return _sc_gather


MB = 4096


def _mlp_body(ueT_ref, ieT_ref, w1_ref, b1_ref, w2_ref, b2_ref,
              wo_ref, bo_ref, out_ref):
    x = jnp.concatenate([ueT_ref[...], ieT_ref[...]], axis=0)
    h = jnp.dot(w1_ref[...], x, preferred_element_type=jnp.float32)
    h = jnp.maximum(h + b1_ref[...], 0.0)
    h = jnp.dot(w2_ref[...], h, preferred_element_type=jnp.float32)
    h = jnp.maximum(h + b2_ref[...], 0.0)
    out_ref[...] = jnp.dot(wo_ref[...], h, preferred_element_type=jnp.float32) + bo_ref[...]


def kernel(user_indices, item_indices, user_emb, item_emb, W1, b1, W2, b2, Wo, bo):
    gather = _make_sc_gather()
    uflat = _restage(user_emb.T).reshape(FLAT)
    ueT = gather(user_indices.astype(jnp.int32), uflat)
    iflat = _restage(item_emb.T).reshape(FLAT)
    ieT = gather(item_indices.astype(jnp.int32), iflat)
    outT = pl.pallas_call(
        _mlp_body,
        grid=(BATCH // MB,),
        in_specs=[
            pl.BlockSpec((EMBED, MB), lambda j: (0, j)),
            pl.BlockSpec((EMBED, MB), lambda j: (0, j)),
            pl.BlockSpec(W1.shape, lambda j: (0, 0)),
            pl.BlockSpec((W1.shape[0], 1), lambda j: (0, 0)),
            pl.BlockSpec(W2.shape, lambda j: (0, 0)),
            pl.BlockSpec((W2.shape[0], 1), lambda j: (0, 0)),
            pl.BlockSpec(Wo.shape, lambda j: (0, 0)),
            pl.BlockSpec((1, 1), lambda j: (0, 0)),
        ],
        out_specs=pl.BlockSpec((1, MB), lambda j: (0, j)),
        out_shape=jax.ShapeDtypeStruct((1, BATCH), jnp.float32),
    )(ueT, ieT, W1, b1.reshape(-1, 1),
      W2, b2.reshape(-1, 1), Wo, bo.reshape(1, 1))
    return outT.reshape(BATCH, 1)


# speedup vs baseline: 1.0082x; 1.0082x over previous
import functools

import jax
import jax.numpy as jnp
from jax import lax
from jax.experimental import pallas as pl
from jax.experimental.pallas import tpu as pltpu
from jax.experimental.pallas import tpu_sc as plsc

BATCH = 16384
EMBED = 32
ROWS = 1000000
DPAD = 7840
STRIDE = DPAD * 128
FLAT = EMBED * STRIDE
NC = 2
NS = 16
NW = NC * NS
BPW = BATCH // NW
CHUNK = 128
NCHUNK = BPW // CHUNK
CB = 32768


def _restage_body(inT_ref, out_ref):
    out_ref[...] = inT_ref[...].reshape(EMBED, CB // 128, 128)


def _restage(tT):
    return pl.pallas_call(
        _restage_body,
        grid=(pl.cdiv(ROWS, CB),),
        in_specs=[pl.BlockSpec((EMBED, CB), lambda j: (0, j))],
        out_specs=pl.BlockSpec((EMBED, CB // 128, 128), lambda j: (0, j, 0)),
        out_shape=jax.ShapeDtypeStruct((EMBED, DPAD, 128), jnp.float32),
    )(tT)


@functools.lru_cache(maxsize=1)
def _make_sc_gather():
    mesh = plsc.VectorSubcoreMesh(core_axis_name="c", subcore_axis_name="s")

    @functools.partial(
        pl.kernel,
        mesh=mesh,
        compiler_params=pltpu.CompilerParams(use_tc_tiling_on_sc=False),
        out_type=jax.ShapeDtypeStruct((EMBED, BATCH), jnp.float32),
        scratch_types=[
            pltpu.VMEM((BPW,), jnp.int32),
            pltpu.VMEM((EMBED, BPW), jnp.float32),
            pltpu.SemaphoreType.DMA,
        ],
    )
    def _sc_gather(idx_hbm, flat2d_hbm, eT_out, idx_v, rowsT_v, sem):
        wid = lax.axis_index("s") * NC + lax.axis_index("c")
        base = wid * BPW
        pltpu.sync_copy(idx_hbm.at[pl.ds(base, BPW)], idx_v)

        copies = []
        for d in range(EMBED):
            for c in range(NCHUNK):
                sl = pl.ds(c * CHUNK, CHUNK)
                copies.append(pltpu.async_copy(
                    flat2d_hbm.at[d].at[idx_v.at[sl]], rowsT_v.at[d, sl], sem))
        for cp in copies:
            cp.wait()

        pltpu.sync_copy(rowsT_v, eT_out.at[:, pl.ds(base, BPW)])

    return _sc_gather


MB = 4096


def _mlp_body(ueT_ref, ieT_ref, w1_ref, b1_ref, w2_ref, b2_ref,
              wo_ref, bo_ref, out_ref):
    x = jnp.concatenate([ueT_ref[...], ieT_ref[...]], axis=0)
    h = jnp.dot(w1_ref[...], x, preferred_element_type=jnp.float32)
    h = jnp.maximum(h + b1_ref[...], 0.0)
    h = jnp.dot(w2_ref[...], h, preferred_element_type=jnp.float32)
    h = jnp.maximum(h + b2_ref[...], 0.0)
    out_ref[...] = jnp.dot(wo_ref[...], h, preferred_element_type=jnp.float32) + bo_ref[...]


def kernel(user_indices, item_indices, user_emb, item_emb, W1, b1, W2, b2, Wo, bo):
    gather = _make_sc_gather()
    uflat = _restage(user_emb.T).reshape(EMBED, STRIDE)
    ueT = gather(user_indices.astype(jnp.int32), uflat)
    iflat = _restage(item_emb.T).reshape(EMBED, STRIDE)
    ieT = gather(item_indices.astype(jnp.int32), iflat)
    outT = pl.pallas_call(
        _mlp_body,
        grid=(BATCH // MB,),
        in_specs=[
            pl.BlockSpec((EMBED, MB), lambda j: (0, j)),
            pl.BlockSpec((EMBED, MB), lambda j: (0, j)),
            pl.BlockSpec(W1.shape, lambda j: (0, 0)),
            pl.BlockSpec((W1.shape[0], 1), lambda j: (0, 0)),
            pl.BlockSpec(W2.shape, lambda j: (0, 0)),
            pl.BlockSpec((W2.shape[0], 1), lambda j: (0, 0)),
            pl.BlockSpec(Wo.shape, lambda j: (0, 0)),
            pl.BlockSpec((1, 1), lambda j: (0, 0)),
        ],
        out_specs=pl.BlockSpec((1, MB), lambda j: (0, j)),
        out_shape=jax.ShapeDtypeStruct((1, BATCH), jnp.float32),
    )(ueT, ieT, W1, b1.reshape(-1, 1),
      W2, b2.reshape(-1, 1), Wo, bo.reshape(1, 1))
    return outT.reshape(BATCH, 1)
